# Initial kernel scaffold; baseline (speedup 1.0000x reference)
import jax
import jax.numpy as jnp
from jax.experimental import pallas as pl


def kernel(x, edge_attr, params, edge_index):
    raise NotImplementedError("write your pallas kernel here")


# speedup vs baseline: 3.0494x; 3.0494x over previous
import functools

import jax
import jax.numpy as jnp
from jax import lax
from jax.experimental import pallas as pl
from jax.experimental.pallas import tpu as pltpu
from jax.experimental.pallas import tpu_sc as plsc

N_NODES = 10000
N_EDGES = 160000
D = 128

NC = 2
NS = 16
NW = NC * NS
EPW = N_EDGES // NW
CHUNK = 40
NCHUNK = EPW // CHUNK
NPAD = 10240
RPT = NPAD // NS

NODE_BLK = 1000
EDGE_BLK = 2000

f32 = jnp.float32


def _ln(h, g, be):
    mu = jnp.mean(h, axis=-1, keepdims=True)
    d = h - mu
    var = jnp.mean(d * d, axis=-1, keepdims=True)
    return d * lax.rsqrt(var + 1e-5) * g + be


def _mlp_ln_body(x_ref, w1, b1, w2, b2, w3, b3, g, be, out_ref):
    h = jnp.maximum(jnp.dot(x_ref[...], w1[...], preferred_element_type=f32) + b1[...], 0.0)
    h = jnp.maximum(jnp.dot(h, w2[...], preferred_element_type=f32) + b2[...], 0.0)
    h = jnp.dot(h, w3[...], preferred_element_type=f32) + b3[...]
    out_ref[...] = _ln(h, g[...], be[...])


def _proj_body(h_ref, ws, wd, ts_ref, td_ref):
    h = h_ref[...]
    ts_ref[...] = jnp.dot(h, ws[...], preferred_element_type=f32)
    td_ref[...] = jnp.dot(h, wd[...], preferred_element_type=f32)


def _edge_body(e_ref, gs_ref, gd_ref, w1e, b1, w2, b2, w3, b3, g, be,
               eupd_ref, enew_ref):
    e = e_ref[...]
    h = jnp.dot(e, w1e[...], preferred_element_type=f32)
    h = jnp.maximum(h + gs_ref[...] + gd_ref[...] + b1[...], 0.0)
    h = jnp.maximum(jnp.dot(h, w2[...], preferred_element_type=f32) + b2[...], 0.0)
    h = jnp.dot(h, w3[...], preferred_element_type=f32) + b3[...]
    u = _ln(h, g[...], be[...])
    eupd_ref[...] = u
    enew_ref[...] = e + u


def _node_body(h_ref, p0_ref, p1_ref, wh, wa, b1, w2, b2, w3, b3, g, be,
               out_ref):
    h = h_ref[...]
    agg = p0_ref[...] + p1_ref[...]
    z = jnp.dot(h, wh[...], preferred_element_type=f32)
    z = z + jnp.dot(agg, wa[...], preferred_element_type=f32)
    z = jnp.maximum(z + b1[...], 0.0)
    z = jnp.maximum(jnp.dot(z, w2[...], preferred_element_type=f32) + b2[...], 0.0)
    z = jnp.dot(z, w3[...], preferred_element_type=f32) + b3[...]
    out_ref[...] = h + _ln(z, g[...], be[...])


def _dec_body(h_ref, w1, b1, w2, b2, w3, b3, out_ref):
    z = jnp.maximum(jnp.dot(h_ref[...], w1[...], preferred_element_type=f32) + b1[...], 0.0)
    z = jnp.maximum(jnp.dot(z, w2[...], preferred_element_type=f32) + b2[...], 0.0)
    out_ref[...] = jnp.dot(z, w3[...], preferred_element_type=f32) + b3[...]


def _row_call(body, nrows, blk, ins, out_count):
    def spec(a):
        nd = a.ndim
        if a.shape[0] >= nrows:
            return pl.BlockSpec((blk,) + a.shape[1:],
                                lambda i, nd=nd: (i,) + (0,) * (nd - 1))
        return pl.BlockSpec(a.shape, lambda i, nd=nd: (0,) * nd)

    out_sh = jax.ShapeDtypeStruct((nrows, D), f32)
    out_spec = pl.BlockSpec((blk, D), lambda i: (i, 0))
    return pl.pallas_call(
        body,
        grid=(nrows // blk,),
        in_specs=[spec(a) for a in ins],
        out_specs=[out_spec] * out_count if out_count > 1 else out_spec,
        out_shape=[out_sh] * out_count if out_count > 1 else out_sh,
    )(*ins)


def _gather_call(tsrc, tdst, src3, dst3):
    mesh = plsc.VectorSubcoreMesh(core_axis_name="c", subcore_axis_name="s")

    @functools.partial(
        pl.kernel,
        out_type=(jax.ShapeDtypeStruct((N_EDGES, D), f32),
                  jax.ShapeDtypeStruct((N_EDGES, D), f32)),
        mesh=mesh,
        scratch_types=[
            pltpu.VMEM((NCHUNK, CHUNK), jnp.int32),
            pltpu.VMEM((NCHUNK, CHUNK), jnp.int32),
            pltpu.VMEM((CHUNK, D), f32),
            pltpu.VMEM((CHUNK, D), f32),
            pltpu.SemaphoreType.DMA,
            pltpu.SemaphoreType.DMA,
        ],
    )
    def k(tsrc_h, tdst_h, src_h, dst_h, gs_h, gd_h,
          si_v, di_v, rs_v, rd_v, sem1, sem2):
        cid = lax.axis_index("c")
        sid = lax.axis_index("s")
        wid = sid * NC + cid
        pltpu.sync_copy(src_h.at[wid], si_v)
        pltpu.sync_copy(dst_h.at[wid], di_v)
        base = wid * EPW

        def chunk(j, carry):
            c1 = pltpu.async_copy(tsrc_h.at[si_v.at[j]], rs_v, sem1)
            c2 = pltpu.async_copy(tdst_h.at[di_v.at[j]], rd_v, sem2)
            c1.wait()
            pltpu.sync_copy(rs_v, gs_h.at[pl.ds(base + j * CHUNK, CHUNK)])
            c2.wait()
            pltpu.sync_copy(rd_v, gd_h.at[pl.ds(base + j * CHUNK, CHUNK)])
            return carry

        lax.fori_loop(0, NCHUNK, chunk, 0)

    return k(tsrc, tdst, src3, dst3)


def _scatter_call(eupd, dst3, zeros):
    mesh = plsc.VectorSubcoreMesh(core_axis_name="c", subcore_axis_name="s")

    @functools.partial(
        pl.kernel,
        out_type=jax.ShapeDtypeStruct((NC, NPAD, D), f32),
        mesh=mesh,
        scratch_types=[
            pltpu.VMEM((NCHUNK, CHUNK), jnp.int32),
            pltpu.VMEM((CHUNK, D), f32),
            pltpu.VMEM_SHARED((NPAD, D), f32),
            pltpu.SemaphoreType.DMA,
        ],
    )
    def k(e_h, dst_h, z_h, parts_h, di_v, rows_v, acc_s, sem):
        cid = lax.axis_index("c")
        sid = lax.axis_index("s")
        wid = sid * NC + cid
        pltpu.sync_copy(z_h.at[pl.ds(sid * RPT, RPT)],
                        acc_s.at[pl.ds(sid * RPT, RPT)])
        pltpu.sync_copy(dst_h.at[wid], di_v)
        plsc.subcore_barrier()
        base = wid * EPW

        def chunk(j, carry):
            pltpu.sync_copy(e_h.at[pl.ds(base + j * CHUNK, CHUNK)], rows_v)
            pltpu.sync_copy(rows_v, acc_s.at[di_v.at[j]], add=True)
            return carry

        lax.fori_loop(0, NCHUNK, chunk, 0)
        plsc.subcore_barrier()
        pltpu.sync_copy(acc_s.at[pl.ds(sid * RPT, RPT)],
                        parts_h.at[cid, pl.ds(sid * RPT, RPT)])

    return k(eupd, dst3, zeros)


def _unpack(layers, ln):
    lin = layers[:-1] if ln else layers
    ws = []
    for W, b in lin:
        ws += [W, b.reshape(1, -1)]
    if ln:
        g, be = layers[-1]
        ws += [g.reshape(1, -1), be.reshape(1, -1)]
    return ws


def kernel(x, edge_attr, params, edge_index):
    src3 = edge_index[0].reshape(NW, NCHUNK, CHUNK)
    dst3 = edge_index[1].reshape(NW, NCHUNK, CHUNK)
    zeros = jnp.zeros((NPAD, D), f32)

    h = _row_call(_mlp_ln_body, N_NODES, NODE_BLK,
                  [x] + _unpack(params['enc_node'], True), 1)
    e = _row_call(_mlp_ln_body, N_EDGES, EDGE_BLK,
                  [edge_attr] + _unpack(params['enc_edge'], True), 1)

    for step in params['proc']:
        (W1, b1), (W2, b2), (W3, b3) = step['edge'][:3]
        g, be = step['edge'][3]
        W1e, W1s, W1d = W1[:D], W1[D:2 * D], W1[2 * D:]
        ts, td = _row_call(_proj_body, N_NODES, NODE_BLK, [h, W1s, W1d], 2)
        gs, gd = _gather_call(ts, td, src3, dst3)
        e_upd, e = _row_call(
            _edge_body, N_EDGES, EDGE_BLK,
            [e, gs, gd, W1e, b1.reshape(1, -1), W2, b2.reshape(1, -1),
             W3, b3.reshape(1, -1), g.reshape(1, -1), be.reshape(1, -1)], 2)
        parts = _scatter_call(e_upd, dst3, zeros)
        (Wn1, nb1), (Wn2, nb2), (Wn3, nb3) = step['node'][:3]
        ng, nbe = step['node'][3]
        h = _row_call(
            _node_body, N_NODES, NODE_BLK,
            [h, parts[0], parts[1], Wn1[:D], Wn1[D:], nb1.reshape(1, -1),
             Wn2, nb2.reshape(1, -1), Wn3, nb3.reshape(1, -1),
             ng.reshape(1, -1), nbe.reshape(1, -1)], 1)

    (dW1, db1), (dW2, db2), (dW3, db3) = params['dec']
    dW3p = jnp.zeros((D, D), f32).at[:, :3].set(dW3)
    db3p = jnp.zeros((1, D), f32).at[0, :3].set(db3)
    out = _row_call(_dec_body, N_NODES, NODE_BLK,
                    [h, dW1, db1.reshape(1, -1), dW2, db2.reshape(1, -1),
                     dW3p, db3p], 1)
    return out[:, :3]


# speedup vs baseline: 3.8478x; 1.2618x over previous
import functools

import jax
import jax.numpy as jnp
from jax import lax
from jax.experimental import pallas as pl
from jax.experimental.pallas import tpu as pltpu
from jax.experimental.pallas import tpu_sc as plsc

N_NODES = 10000
N_EDGES = 160000
D = 128

NC = 2
NS = 16
NW = NC * NS
EPW = N_EDGES // NW
CHUNK = 40
NCHUNK = EPW // CHUNK
SUB = 5
GROUP = SUB * CHUNK
NGROUP = EPW // GROUP
NPAD = 10240
RPT = NPAD // NS

NODE_BLK = 1000
EDGE_BLK = 2000

f32 = jnp.float32


def _ln(h, g, be):
    mu = jnp.mean(h, axis=-1, keepdims=True)
    d = h - mu
    var = jnp.mean(d * d, axis=-1, keepdims=True)
    return d * lax.rsqrt(var + 1e-5) * g + be


def _mlp_ln_body(x_ref, w1, b1, w2, b2, w3, b3, g, be, out_ref):
    h = jnp.maximum(jnp.dot(x_ref[...], w1[...], preferred_element_type=f32) + b1[...], 0.0)
    h = jnp.maximum(jnp.dot(h, w2[...], preferred_element_type=f32) + b2[...], 0.0)
    h = jnp.dot(h, w3[...], preferred_element_type=f32) + b3[...]
    out_ref[...] = _ln(h, g[...], be[...])


def _proj_body(h_ref, ws, wd, ts_ref, td_ref):
    h = h_ref[...]
    ts_ref[...] = jnp.dot(h, ws[...], preferred_element_type=f32)
    td_ref[...] = jnp.dot(h, wd[...], preferred_element_type=f32)


def _edge_body(e_ref, gs_ref, gd_ref, w1e, b1, w2, b2, w3, b3, g, be,
               eupd_ref, enew_ref):
    e = e_ref[...]
    h = jnp.dot(e, w1e[...], preferred_element_type=f32)
    h = jnp.maximum(h + gs_ref[...] + gd_ref[...] + b1[...], 0.0)
    h = jnp.maximum(jnp.dot(h, w2[...], preferred_element_type=f32) + b2[...], 0.0)
    h = jnp.dot(h, w3[...], preferred_element_type=f32) + b3[...]
    u = _ln(h, g[...], be[...])
    eupd_ref[...] = u
    enew_ref[...] = e + u


def _node_body(h_ref, p0_ref, p1_ref, wh, wa, b1, w2, b2, w3, b3, g, be,
               out_ref):
    h = h_ref[...]
    agg = p0_ref[...] + p1_ref[...]
    z = jnp.dot(h, wh[...], preferred_element_type=f32)
    z = z + jnp.dot(agg, wa[...], preferred_element_type=f32)
    z = jnp.maximum(z + b1[...], 0.0)
    z = jnp.maximum(jnp.dot(z, w2[...], preferred_element_type=f32) + b2[...], 0.0)
    z = jnp.dot(z, w3[...], preferred_element_type=f32) + b3[...]
    out_ref[...] = h + _ln(z, g[...], be[...])


def _dec_body(h_ref, w1, b1, w2, b2, w3, b3, out_ref):
    z = jnp.maximum(jnp.dot(h_ref[...], w1[...], preferred_element_type=f32) + b1[...], 0.0)
    z = jnp.maximum(jnp.dot(z, w2[...], preferred_element_type=f32) + b2[...], 0.0)
    out_ref[...] = jnp.dot(z, w3[...], preferred_element_type=f32) + b3[...]


def _row_call(body, nrows, blk, ins, out_count):
    def spec(a):
        nd = a.ndim
        if a.shape[0] >= nrows:
            return pl.BlockSpec((blk,) + a.shape[1:],
                                lambda i, nd=nd: (i,) + (0,) * (nd - 1))
        return pl.BlockSpec(a.shape, lambda i, nd=nd: (0,) * nd)

    out_sh = jax.ShapeDtypeStruct((nrows, D), f32)
    out_spec = pl.BlockSpec((blk, D), lambda i: (i, 0))
    return pl.pallas_call(
        body,
        grid=(nrows // blk,),
        in_specs=[spec(a) for a in ins],
        out_specs=[out_spec] * out_count if out_count > 1 else out_spec,
        out_shape=[out_sh] * out_count if out_count > 1 else out_sh,
    )(*ins)


def _gather_call(tsrc, tdst, src3, dst3):
    mesh = plsc.VectorSubcoreMesh(core_axis_name="c", subcore_axis_name="s")

    @functools.partial(
        pl.kernel,
        out_type=(jax.ShapeDtypeStruct((N_EDGES, D), f32),
                  jax.ShapeDtypeStruct((N_EDGES, D), f32)),
        mesh=mesh,
        scratch_types=[
            pltpu.VMEM((NCHUNK, CHUNK), jnp.int32),
            pltpu.VMEM((NCHUNK, CHUNK), jnp.int32),
            pltpu.VMEM((CHUNK, D), f32),
            pltpu.VMEM((CHUNK, D), f32),
            pltpu.VMEM((CHUNK, D), f32),
            pltpu.VMEM((CHUNK, D), f32),
            pltpu.SemaphoreType.DMA,
            pltpu.SemaphoreType.DMA,
            pltpu.SemaphoreType.DMA,
            pltpu.SemaphoreType.DMA,
            pltpu.SemaphoreType.DMA,
            pltpu.SemaphoreType.DMA,
            pltpu.SemaphoreType.DMA,
            pltpu.SemaphoreType.DMA,
        ],
    )
    def k(tsrc_h, tdst_h, src_h, dst_h, gs_h, gd_h,
          si_v, di_v, rs0, rs1, rd0, rd1,
          gsem0, gsem1, dsem0, dsem1, ss0, ss1, sd0, sd1):
        cid = lax.axis_index("c")
        sid = lax.axis_index("s")
        wid = sid * NC + cid
        pltpu.sync_copy(src_h.at[wid], si_v)
        pltpu.sync_copy(dst_h.at[wid], di_v)
        base = wid * EPW

        def fire(tbl, idxv, buf, sem, g):
            pltpu.async_copy(tbl.at[idxv.at[g]], buf, sem)

        def drain(tbl, buf, sem):
            pltpu.make_async_copy(tbl.at[pl.ds(0, CHUNK)], buf, sem).wait()

        def store(buf, out, sem, g):
            pltpu.async_copy(buf, out.at[pl.ds(base + g * CHUNK, CHUNK)], sem)

        def store_wait(buf, out, sem):
            pltpu.make_async_copy(buf, out.at[pl.ds(0, CHUNK)], sem).wait()

        fire(tsrc_h, si_v, rs0, gsem0, 0)
        fire(tdst_h, di_v, rd0, dsem0, 0)
        fire(tsrc_h, si_v, rs1, gsem1, 1)
        fire(tdst_h, di_v, rd1, dsem1, 1)

        def stage(tbl, idxv, buf, gsem, out, ssem, g, nxt):
            drain(tbl, buf, gsem)
            store(buf, out, ssem, g)
            store_wait(buf, out, ssem)
            if nxt is not None:
                fire(tbl, idxv, buf, gsem, nxt)

        def body(j, carry):
            g = 2 * j
            stage(tsrc_h, si_v, rs0, gsem0, gs_h, ss0, g, g + 2)
            stage(tdst_h, di_v, rd0, dsem0, gd_h, sd0, g, g + 2)
            stage(tsrc_h, si_v, rs1, gsem1, gs_h, ss1, g + 1, g + 3)
            stage(tdst_h, di_v, rd1, dsem1, gd_h, sd1, g + 1, g + 3)
            return carry

        lax.fori_loop(0, (NCHUNK - 3) // 2, body, 0)
        stage(tsrc_h, si_v, rs0, gsem0, gs_h, ss0, NCHUNK - 3, NCHUNK - 1)
        stage(tdst_h, di_v, rd0, dsem0, gd_h, sd0, NCHUNK - 3, NCHUNK - 1)
        stage(tsrc_h, si_v, rs1, gsem1, gs_h, ss1, NCHUNK - 2, None)
        stage(tdst_h, di_v, rd1, dsem1, gd_h, sd1, NCHUNK - 2, None)
        stage(tsrc_h, si_v, rs0, gsem0, gs_h, ss0, NCHUNK - 1, None)
        stage(tdst_h, di_v, rd0, dsem0, gd_h, sd0, NCHUNK - 1, None)

    return k(tsrc, tdst, src3, dst3)


def _scatter_call(eupd, dst3, zeros):
    mesh = plsc.VectorSubcoreMesh(core_axis_name="c", subcore_axis_name="s")

    @functools.partial(
        pl.kernel,
        out_type=jax.ShapeDtypeStruct((NC, NPAD, D), f32),
        mesh=mesh,
        scratch_types=[
            pltpu.VMEM((NCHUNK, CHUNK), jnp.int32),
            pltpu.VMEM((CHUNK, D), f32),
            pltpu.VMEM((CHUNK, D), f32),
            pltpu.VMEM_SHARED((NPAD, D), f32),
            pltpu.SemaphoreType.DMA,
            pltpu.SemaphoreType.DMA,
            pltpu.SemaphoreType.DMA,
            pltpu.SemaphoreType.DMA,
        ],
    )
    def k(e_h, dst_h, z_h, parts_h, di_v, eb0, eb1, acc_s,
          lsem0, lsem1, asem0, asem1):
        cid = lax.axis_index("c")
        sid = lax.axis_index("s")
        wid = sid * NC + cid
        pltpu.sync_copy(z_h.at[pl.ds(sid * RPT, RPT)],
                        acc_s.at[pl.ds(sid * RPT, RPT)])
        pltpu.sync_copy(dst_h.at[wid], di_v)
        plsc.subcore_barrier()
        base = wid * EPW

        def load(buf, sem, g):
            pltpu.async_copy(e_h.at[pl.ds(base + g * CHUNK, CHUNK)], buf, sem)

        def load_wait(buf, sem):
            pltpu.make_async_copy(e_h.at[pl.ds(0, CHUNK)], buf, sem).wait()

        def add(buf, sem, g):
            pltpu.async_copy(buf, acc_s.at[di_v.at[g]], sem, add=True)

        def add_wait(buf, sem):
            pltpu.make_async_copy(buf, acc_s.at[pl.ds(0, CHUNK)], sem).wait()

        load(eb0, lsem0, 0)
        load(eb1, lsem1, 1)

        def body(j, carry):
            g = 2 * j
            load_wait(eb0, lsem0)
            add(eb0, asem0, g)
            add_wait(eb0, asem0)
            load(eb0, lsem0, g + 2)
            load_wait(eb1, lsem1)
            add(eb1, asem1, g + 1)
            add_wait(eb1, asem1)
            load(eb1, lsem1, g + 3)
            return carry

        lax.fori_loop(0, (NCHUNK - 3) // 2, body, 0)
        load_wait(eb0, lsem0)
        add(eb0, asem0, NCHUNK - 3)
        add_wait(eb0, asem0)
        load(eb0, lsem0, NCHUNK - 1)
        load_wait(eb1, lsem1)
        add(eb1, asem1, NCHUNK - 2)
        add_wait(eb1, asem1)
        load_wait(eb0, lsem0)
        add(eb0, asem0, NCHUNK - 1)
        add_wait(eb0, asem0)
        plsc.subcore_barrier()
        pltpu.sync_copy(acc_s.at[pl.ds(sid * RPT, RPT)],
                        parts_h.at[cid, pl.ds(sid * RPT, RPT)])

    return k(eupd, dst3, zeros)


def _unpack(layers, ln):
    lin = layers[:-1] if ln else layers
    ws = []
    for W, b in lin:
        ws += [W, b.reshape(1, -1)]
    if ln:
        g, be = layers[-1]
        ws += [g.reshape(1, -1), be.reshape(1, -1)]
    return ws


def kernel(x, edge_attr, params, edge_index):
    src3 = edge_index[0].reshape(NW, NCHUNK, CHUNK)
    dst3 = edge_index[1].reshape(NW, NCHUNK, CHUNK)
    zeros = jnp.zeros((NPAD, D), f32)

    h = _row_call(_mlp_ln_body, N_NODES, NODE_BLK,
                  [x] + _unpack(params['enc_node'], True), 1)
    e = _row_call(_mlp_ln_body, N_EDGES, EDGE_BLK,
                  [edge_attr] + _unpack(params['enc_edge'], True), 1)

    for step in params['proc']:
        (W1, b1), (W2, b2), (W3, b3) = step['edge'][:3]
        g, be = step['edge'][3]
        W1e, W1s, W1d = W1[:D], W1[D:2 * D], W1[2 * D:]
        ts, td = _row_call(_proj_body, N_NODES, NODE_BLK, [h, W1s, W1d], 2)
        gs, gd = _gather_call(ts, td, src3, dst3)
        e_upd, e = _row_call(
            _edge_body, N_EDGES, EDGE_BLK,
            [e, gs, gd, W1e, b1.reshape(1, -1), W2, b2.reshape(1, -1),
             W3, b3.reshape(1, -1), g.reshape(1, -1), be.reshape(1, -1)], 2)
        parts = _scatter_call(e_upd, dst3, zeros)
        (Wn1, nb1), (Wn2, nb2), (Wn3, nb3) = step['node'][:3]
        ng, nbe = step['node'][3]
        h = _row_call(
            _node_body, N_NODES, NODE_BLK,
            [h, parts[0], parts[1], Wn1[:D], Wn1[D:], nb1.reshape(1, -1),
             Wn2, nb2.reshape(1, -1), Wn3, nb3.reshape(1, -1),
             ng.reshape(1, -1), nbe.reshape(1, -1)], 1)

    (dW1, db1), (dW2, db2), (dW3, db3) = params['dec']
    dW3p = jnp.zeros((D, D), f32).at[:, :3].set(dW3)
    db3p = jnp.zeros((1, D), f32).at[0, :3].set(db3)
    out = _row_call(_dec_body, N_NODES, NODE_BLK,
                    [h, dW1, db1.reshape(1, -1), dW2, db2.reshape(1, -1),
                     dW3p, db3p], 1)
    return out[:, :3]


# speedup vs baseline: 4.2082x; 1.0937x over previous
import functools

import jax
import jax.numpy as jnp
from jax import lax
from jax.experimental import pallas as pl
from jax.experimental.pallas import tpu as pltpu
from jax.experimental.pallas import tpu_sc as plsc

N_NODES = 10000
N_EDGES = 160000
D = 128

NC = 2
NS = 16
NW = NC * NS
CHUNK = 40
NPAD = 10240
RPT = NPAD // NS

NCA = 62
NCB = 63
EA = NW * NCA * CHUNK
EB = NW * NCB * CHUNK

NODE_BLK = 1000
BLK_A = EA // 32
BLK_B = EB // 32

f32 = jnp.float32


def _ln(h, g, be):
    mu = jnp.mean(h, axis=-1, keepdims=True)
    d = h - mu
    var = jnp.mean(d * d, axis=-1, keepdims=True)
    return d * lax.rsqrt(var + 1e-5) * g + be


def _mlp_ln_body(x_ref, w1, b1, w2, b2, w3, b3, g, be, out_ref):
    h = jnp.maximum(jnp.dot(x_ref[...], w1[...], preferred_element_type=f32) + b1[...], 0.0)
    h = jnp.maximum(jnp.dot(h, w2[...], preferred_element_type=f32) + b2[...], 0.0)
    h = jnp.dot(h, w3[...], preferred_element_type=f32) + b3[...]
    out_ref[...] = _ln(h, g[...], be[...])


def _proj_body(h_ref, ws, wd, ts_ref, td_ref):
    h = h_ref[...]
    ts_ref[...] = jnp.dot(h, ws[...], preferred_element_type=f32)
    td_ref[...] = jnp.dot(h, wd[...], preferred_element_type=f32)


def _edge_mlp(e, gs, gd, w1e, b1, w2, b2, w3, b3, g, be):
    h = jnp.dot(e, w1e[...], preferred_element_type=f32)
    h = jnp.maximum(h + gs + gd + b1[...], 0.0)
    h = jnp.maximum(jnp.dot(h, w2[...], preferred_element_type=f32) + b2[...], 0.0)
    h = jnp.dot(h, w3[...], preferred_element_type=f32) + b3[...]
    return _ln(h, g[...], be[...])


def _edge_body(e_ref, gs_ref, gd_ref, w1e, b1, w2, b2, w3, b3, g, be,
               eupd_ref, enew_ref):
    e = e_ref[...]
    u = _edge_mlp(e, gs_ref[...], gd_ref[...], w1e, b1, w2, b2, w3, b3, g, be)
    eupd_ref[...] = u
    enew_ref[...] = e + u


def _edge_body_last(e_ref, gs_ref, gd_ref, w1e, b1, w2, b2, w3, b3, g, be,
                    eupd_ref):
    u = _edge_mlp(e_ref[...], gs_ref[...], gd_ref[...],
                  w1e, b1, w2, b2, w3, b3, g, be)
    eupd_ref[...] = u


def _node_body(h_ref, pa0_ref, pa1_ref, pb0_ref, pb1_ref,
               wh, wa, b1, w2, b2, w3, b3, g, be, out_ref):
    h = h_ref[...]
    agg = (pa0_ref[...] + pa1_ref[...]) + (pb0_ref[...] + pb1_ref[...])
    z = jnp.dot(h, wh[...], preferred_element_type=f32)
    z = z + jnp.dot(agg, wa[...], preferred_element_type=f32)
    z = jnp.maximum(z + b1[...], 0.0)
    z = jnp.maximum(jnp.dot(z, w2[...], preferred_element_type=f32) + b2[...], 0.0)
    z = jnp.dot(z, w3[...], preferred_element_type=f32) + b3[...]
    out_ref[...] = h + _ln(z, g[...], be[...])


def _dec_body(h_ref, w1, b1, w2, b2, w3, b3, out_ref):
    z = jnp.maximum(jnp.dot(h_ref[...], w1[...], preferred_element_type=f32) + b1[...], 0.0)
    z = jnp.maximum(jnp.dot(z, w2[...], preferred_element_type=f32) + b2[...], 0.0)
    out_ref[...] = jnp.dot(z, w3[...], preferred_element_type=f32) + b3[...]


def _row_call(body, nrows, blk, ins, out_count):
    def spec(a):
        nd = a.ndim
        if a.shape[0] >= nrows:
            return pl.BlockSpec((blk,) + a.shape[1:],
                                lambda i, nd=nd: (i,) + (0,) * (nd - 1))
        return pl.BlockSpec(a.shape, lambda i, nd=nd: (0,) * nd)

    out_sh = jax.ShapeDtypeStruct((nrows, D), f32)
    out_spec = pl.BlockSpec((blk, D), lambda i: (i, 0))
    return pl.pallas_call(
        body,
        grid=(nrows // blk,),
        in_specs=[spec(a) for a in ins],
        out_specs=[out_spec] * out_count if out_count > 1 else out_spec,
        out_shape=[out_sh] * out_count if out_count > 1 else out_sh,
    )(*ins)


def _sc_mesh():
    return plsc.VectorSubcoreMesh(core_axis_name="c", subcore_axis_name="s")


def _gather_call(tsrc, tdst, src3, dst3, nchunk):
    n_edges = NW * nchunk * CHUNK
    epw = nchunk * CHUNK

    @functools.partial(
        pl.kernel,
        out_type=(jax.ShapeDtypeStruct((n_edges, D), f32),
                  jax.ShapeDtypeStruct((n_edges, D), f32)),
        mesh=_sc_mesh(),
        scratch_types=[
            pltpu.VMEM((nchunk, CHUNK), jnp.int32),
            pltpu.VMEM((nchunk, CHUNK), jnp.int32),
            pltpu.VMEM((CHUNK, D), f32),
            pltpu.VMEM((CHUNK, D), f32),
            pltpu.VMEM((CHUNK, D), f32),
            pltpu.VMEM((CHUNK, D), f32),
            pltpu.SemaphoreType.DMA,
            pltpu.SemaphoreType.DMA,
            pltpu.SemaphoreType.DMA,
            pltpu.SemaphoreType.DMA,
            pltpu.SemaphoreType.DMA,
            pltpu.SemaphoreType.DMA,
            pltpu.SemaphoreType.DMA,
            pltpu.SemaphoreType.DMA,
        ],
    )
    def k(tsrc_h, tdst_h, src_h, dst_h, gs_h, gd_h,
          si_v, di_v, rs0, rs1, rd0, rd1,
          gsem0, gsem1, dsem0, dsem1, ss0, ss1, sd0, sd1):
        cid = lax.axis_index("c")
        sid = lax.axis_index("s")
        wid = sid * NC + cid
        pltpu.sync_copy(src_h.at[wid], si_v)
        pltpu.sync_copy(dst_h.at[wid], di_v)
        base = wid * epw

        def fire(tbl, idxv, buf, sem, g):
            pltpu.async_copy(tbl.at[idxv.at[g]], buf, sem)

        def stage(tbl, idxv, buf, gsem, out, ssem, g, nxt):
            pltpu.make_async_copy(tbl.at[pl.ds(0, CHUNK)], buf, gsem).wait()
            pltpu.async_copy(buf, out.at[pl.ds(base + g * CHUNK, CHUNK)], ssem)
            pltpu.make_async_copy(buf, out.at[pl.ds(0, CHUNK)], ssem).wait()
            if nxt is not None:
                fire(tbl, idxv, buf, gsem, nxt)

        fire(tsrc_h, si_v, rs0, gsem0, 0)
        fire(tdst_h, di_v, rd0, dsem0, 0)
        fire(tsrc_h, si_v, rs1, gsem1, 1)
        fire(tdst_h, di_v, rd1, dsem1, 1)

        def body(j, carry):
            g = 2 * j
            stage(tsrc_h, si_v, rs0, gsem0, gs_h, ss0, g, g + 2)
            stage(tdst_h, di_v, rd0, dsem0, gd_h, sd0, g, g + 2)
            stage(tsrc_h, si_v, rs1, gsem1, gs_h, ss1, g + 1, g + 3)
            stage(tdst_h, di_v, rd1, dsem1, gd_h, sd1, g + 1, g + 3)
            return carry

        lax.fori_loop(0, (nchunk - 2) // 2, body, 0)
        if nchunk % 2 == 0:
            stage(tsrc_h, si_v, rs0, gsem0, gs_h, ss0, nchunk - 2, None)
            stage(tdst_h, di_v, rd0, dsem0, gd_h, sd0, nchunk - 2, None)
            stage(tsrc_h, si_v, rs1, gsem1, gs_h, ss1, nchunk - 1, None)
            stage(tdst_h, di_v, rd1, dsem1, gd_h, sd1, nchunk - 1, None)
        else:
            stage(tsrc_h, si_v, rs0, gsem0, gs_h, ss0, nchunk - 3, nchunk - 1)
            stage(tdst_h, di_v, rd0, dsem0, gd_h, sd0, nchunk - 3, nchunk - 1)
            stage(tsrc_h, si_v, rs1, gsem1, gs_h, ss1, nchunk - 2, None)
            stage(tdst_h, di_v, rd1, dsem1, gd_h, sd1, nchunk - 2, None)
            stage(tsrc_h, si_v, rs0, gsem0, gs_h, ss0, nchunk - 1, None)
            stage(tdst_h, di_v, rd0, dsem0, gd_h, sd0, nchunk - 1, None)

    return k(tsrc, tdst, src3, dst3)


def _scatter_call(eupd, dst3, zeros, nchunk):
    epw = nchunk * CHUNK

    @functools.partial(
        pl.kernel,
        out_type=jax.ShapeDtypeStruct((NC, NPAD, D), f32),
        mesh=_sc_mesh(),
        scratch_types=[
            pltpu.VMEM((nchunk, CHUNK), jnp.int32),
            pltpu.VMEM((CHUNK, D), f32),
            pltpu.VMEM((CHUNK, D), f32),
            pltpu.VMEM_SHARED((NPAD, D), f32),
            pltpu.SemaphoreType.DMA,
            pltpu.SemaphoreType.DMA,
            pltpu.SemaphoreType.DMA,
            pltpu.SemaphoreType.DMA,
        ],
    )
    def k(e_h, dst_h, z_h, parts_h, di_v, eb0, eb1, acc_s,
          lsem0, lsem1, asem0, asem1):
        cid = lax.axis_index("c")
        sid = lax.axis_index("s")
        wid = sid * NC + cid
        pltpu.sync_copy(z_h.at[pl.ds(sid * RPT, RPT)],
                        acc_s.at[pl.ds(sid * RPT, RPT)])
        pltpu.sync_copy(dst_h.at[wid], di_v)
        plsc.subcore_barrier()
        base = wid * epw

        def load(buf, sem, g):
            pltpu.async_copy(e_h.at[pl.ds(base + g * CHUNK, CHUNK)], buf, sem)

        def stage(buf, lsem, asem, g, nxt):
            pltpu.make_async_copy(e_h.at[pl.ds(0, CHUNK)], buf, lsem).wait()
            pltpu.async_copy(buf, acc_s.at[di_v.at[g]], asem, add=True)
            pltpu.make_async_copy(buf, acc_s.at[pl.ds(0, CHUNK)], asem).wait()
            if nxt is not None:
                load(buf, lsem, nxt)

        load(eb0, lsem0, 0)
        load(eb1, lsem1, 1)

        def body(j, carry):
            g = 2 * j
            stage(eb0, lsem0, asem0, g, g + 2)
            stage(eb1, lsem1, asem1, g + 1, g + 3)
            return carry

        lax.fori_loop(0, (nchunk - 2) // 2, body, 0)
        if nchunk % 2 == 0:
            stage(eb0, lsem0, asem0, nchunk - 2, None)
            stage(eb1, lsem1, asem1, nchunk - 1, None)
        else:
            stage(eb0, lsem0, asem0, nchunk - 3, nchunk - 1)
            stage(eb1, lsem1, asem1, nchunk - 2, None)
            stage(eb0, lsem0, asem0, nchunk - 1, None)
        plsc.subcore_barrier()
        pltpu.sync_copy(acc_s.at[pl.ds(sid * RPT, RPT)],
                        parts_h.at[cid, pl.ds(sid * RPT, RPT)])

    return k(eupd, dst3, zeros)


def _unpack(layers, ln):
    lin = layers[:-1] if ln else layers
    ws = []
    for W, b in lin:
        ws += [W, b.reshape(1, -1)]
    if ln:
        g, be = layers[-1]
        ws += [g.reshape(1, -1), be.reshape(1, -1)]
    return ws


def kernel(x, edge_attr, params, edge_index):
    src = edge_index[0]
    dst = edge_index[1]
    srcA = src[:EA].reshape(NW, NCA, CHUNK)
    dstA = dst[:EA].reshape(NW, NCA, CHUNK)
    srcB = src[EA:].reshape(NW, NCB, CHUNK)
    dstB = dst[EA:].reshape(NW, NCB, CHUNK)
    zeros = jnp.zeros((NPAD, D), f32)

    h = _row_call(_mlp_ln_body, N_NODES, NODE_BLK,
                  [x] + _unpack(params['enc_node'], True), 1)
    enc_e = _unpack(params['enc_edge'], True)
    eA = _row_call(_mlp_ln_body, EA, BLK_A, [edge_attr[:EA]] + enc_e, 1)
    eB = _row_call(_mlp_ln_body, EB, BLK_B, [edge_attr[EA:]] + enc_e, 1)

    n_steps = len(params['proc'])
    for t, step in enumerate(params['proc']):
        last = t == n_steps - 1
        (W1, b1), (W2, b2), (W3, b3) = step['edge'][:3]
        g, be = step['edge'][3]
        W1e, W1s, W1d = W1[:D], W1[D:2 * D], W1[2 * D:]
        ew = [W1e, b1.reshape(1, -1), W2, b2.reshape(1, -1),
              W3, b3.reshape(1, -1), g.reshape(1, -1), be.reshape(1, -1)]
        ts, td = _row_call(_proj_body, N_NODES, NODE_BLK, [h, W1s, W1d], 2)
        gsA, gdA = _gather_call(ts, td, srcA, dstA, NCA)
        gsB, gdB = _gather_call(ts, td, srcB, dstB, NCB)
        if last:
            uA = _row_call(_edge_body_last, EA, BLK_A, [eA, gsA, gdA] + ew, 1)
            uB = _row_call(_edge_body_last, EB, BLK_B, [eB, gsB, gdB] + ew, 1)
        else:
            uA, eA = _row_call(_edge_body, EA, BLK_A, [eA, gsA, gdA] + ew, 2)
            uB, eB = _row_call(_edge_body, EB, BLK_B, [eB, gsB, gdB] + ew, 2)
        partsA = _scatter_call(uA, dstA, zeros, NCA)
        partsB = _scatter_call(uB, dstB, zeros, NCB)
        (Wn1, nb1), (Wn2, nb2), (Wn3, nb3) = step['node'][:3]
        ng, nbe = step['node'][3]
        h = _row_call(
            _node_body, N_NODES, NODE_BLK,
            [h, partsA[0], partsA[1], partsB[0], partsB[1],
             Wn1[:D], Wn1[D:], nb1.reshape(1, -1),
             Wn2, nb2.reshape(1, -1), Wn3, nb3.reshape(1, -1),
             ng.reshape(1, -1), nbe.reshape(1, -1)], 1)

    (dW1, db1), (dW2, db2), (dW3, db3) = params['dec']
    dW3p = jnp.zeros((D, D), f32).at[:, :3].set(dW3)
    db3p = jnp.zeros((1, D), f32).at[0, :3].set(db3)
    out = _row_call(_dec_body, N_NODES, NODE_BLK,
                    [h, dW1, db1.reshape(1, -1), dW2, db2.reshape(1, -1),
                     dW3p, db3p], 1)
    return out[:, :3]


# speedup vs baseline: 4.3066x; 1.0234x over previous
import functools

import jax
import jax.numpy as jnp
from jax import lax
from jax.experimental import pallas as pl
from jax.experimental.pallas import tpu as pltpu
from jax.experimental.pallas import tpu_sc as plsc

N_NODES = 10000
N_EDGES = 160000
D = 128

NC = 2
NS = 16
NW = NC * NS
BIG = 128
NMAIN = 19
NPAD = 10240
RPT = NPAD // NS

EPWA = 2480
EPWB = 2520
TAILA = EPWA - NMAIN * BIG
TAILB = EPWB - NMAIN * BIG
EA = NW * EPWA
EB = NW * EPWB

NODE_BLK = 1000
BLK_A = EA // 32
BLK_B = EB // 32

f32 = jnp.float32


def _ln(h, g, be):
    mu = jnp.mean(h, axis=-1, keepdims=True)
    d = h - mu
    var = jnp.mean(d * d, axis=-1, keepdims=True)
    return d * lax.rsqrt(var + 1e-5) * g + be


def _mlp_ln_body(x_ref, w1, b1, w2, b2, w3, b3, g, be, out_ref):
    h = jnp.maximum(jnp.dot(x_ref[...], w1[...], preferred_element_type=f32) + b1[...], 0.0)
    h = jnp.maximum(jnp.dot(h, w2[...], preferred_element_type=f32) + b2[...], 0.0)
    h = jnp.dot(h, w3[...], preferred_element_type=f32) + b3[...]
    out_ref[...] = _ln(h, g[...], be[...])


def _proj_body(h_ref, ws, wd, ts_ref, td_ref):
    h = h_ref[...]
    ts_ref[...] = jnp.dot(h, ws[...], preferred_element_type=f32)
    td_ref[...] = jnp.dot(h, wd[...], preferred_element_type=f32)


def _edge_mlp(e, gs, gd, w1e, b1, w2, b2, w3, b3, g, be):
    h = jnp.dot(e, w1e[...], preferred_element_type=f32)
    h = jnp.maximum(h + gs + gd + b1[...], 0.0)
    h = jnp.maximum(jnp.dot(h, w2[...], preferred_element_type=f32) + b2[...], 0.0)
    h = jnp.dot(h, w3[...], preferred_element_type=f32) + b3[...]
    return _ln(h, g[...], be[...])


def _edge_body(e_ref, gs_ref, gd_ref, w1e, b1, w2, b2, w3, b3, g, be,
               eupd_ref, enew_ref):
    e = e_ref[...]
    u = _edge_mlp(e, gs_ref[...], gd_ref[...], w1e, b1, w2, b2, w3, b3, g, be)
    eupd_ref[...] = u
    enew_ref[...] = e + u


def _edge_body_last(e_ref, gs_ref, gd_ref, w1e, b1, w2, b2, w3, b3, g, be,
                    eupd_ref):
    u = _edge_mlp(e_ref[...], gs_ref[...], gd_ref[...],
                  w1e, b1, w2, b2, w3, b3, g, be)
    eupd_ref[...] = u


def _node_body(h_ref, pa0_ref, pa1_ref, pb0_ref, pb1_ref,
               wh, wa, b1, w2, b2, w3, b3, g, be, out_ref):
    h = h_ref[...]
    agg = (pa0_ref[...] + pa1_ref[...]) + (pb0_ref[...] + pb1_ref[...])
    z = jnp.dot(h, wh[...], preferred_element_type=f32)
    z = z + jnp.dot(agg, wa[...], preferred_element_type=f32)
    z = jnp.maximum(z + b1[...], 0.0)
    z = jnp.maximum(jnp.dot(z, w2[...], preferred_element_type=f32) + b2[...], 0.0)
    z = jnp.dot(z, w3[...], preferred_element_type=f32) + b3[...]
    out_ref[...] = h + _ln(z, g[...], be[...])


def _dec_body(h_ref, w1, b1, w2, b2, w3, b3, out_ref):
    z = jnp.maximum(jnp.dot(h_ref[...], w1[...], preferred_element_type=f32) + b1[...], 0.0)
    z = jnp.maximum(jnp.dot(z, w2[...], preferred_element_type=f32) + b2[...], 0.0)
    out_ref[...] = jnp.dot(z, w3[...], preferred_element_type=f32) + b3[...]


def _row_call(body, nrows, blk, ins, out_count, out_dim=D, out_dtype=f32):
    def spec(a):
        nd = a.ndim
        if a.shape[0] >= nrows:
            return pl.BlockSpec((blk,) + a.shape[1:],
                                lambda i, nd=nd: (i,) + (0,) * (nd - 1))
        return pl.BlockSpec(a.shape, lambda i, nd=nd: (0,) * nd)

    out_sh = jax.ShapeDtypeStruct((nrows, out_dim), out_dtype)
    out_spec = pl.BlockSpec((blk, out_dim), lambda i: (i, 0))
    return pl.pallas_call(
        body,
        grid=(nrows // blk,),
        in_specs=[spec(a) for a in ins],
        out_specs=[out_spec] * out_count if out_count > 1 else out_spec,
        out_shape=[out_sh] * out_count if out_count > 1 else out_sh,
    )(*ins)


def _sc_mesh():
    return plsc.VectorSubcoreMesh(core_axis_name="c", subcore_axis_name="s")


def _gather_call(tsrc, tdst, srcm, srct, dstm, dstt, epw, tail):
    n_edges = NW * epw

    @functools.partial(
        pl.kernel,
        out_type=(jax.ShapeDtypeStruct((n_edges, D), f32),
                  jax.ShapeDtypeStruct((n_edges, D), f32)),
        mesh=_sc_mesh(),
        scratch_types=[
            pltpu.VMEM((NMAIN, BIG), jnp.int32),
            pltpu.VMEM((NMAIN, BIG), jnp.int32),
            pltpu.VMEM((1, tail), jnp.int32),
            pltpu.VMEM((1, tail), jnp.int32),
            pltpu.VMEM((BIG, D), f32),
            pltpu.VMEM((BIG, D), f32),
            pltpu.VMEM((BIG, D), f32),
            pltpu.VMEM((BIG, D), f32),
            pltpu.SemaphoreType.DMA,
            pltpu.SemaphoreType.DMA,
            pltpu.SemaphoreType.DMA,
            pltpu.SemaphoreType.DMA,
            pltpu.SemaphoreType.DMA,
            pltpu.SemaphoreType.DMA,
            pltpu.SemaphoreType.DMA,
            pltpu.SemaphoreType.DMA,
        ],
    )
    def k(tsrc_h, tdst_h, srcm_h, srct_h, dstm_h, dstt_h, gs_h, gd_h,
          si_v, di_v, st_v, dt_v, rs0, rs1, rd0, rd1,
          gsem0, gsem1, dsem0, dsem1, ss0, ss1, sd0, sd1):
        cid = lax.axis_index("c")
        sid = lax.axis_index("s")
        wid = sid * NC + cid
        pltpu.sync_copy(srcm_h.at[wid], si_v)
        pltpu.sync_copy(dstm_h.at[wid], di_v)
        pltpu.sync_copy(srct_h.at[wid], st_v)
        pltpu.sync_copy(dstt_h.at[wid], dt_v)
        base = wid * epw

        def fire(tbl, idxv, buf, sem, g):
            pltpu.async_copy(tbl.at[idxv.at[g]], buf, sem)

        def stage(tbl, idxv, buf, gsem, out, ssem, g, nxt):
            pltpu.make_async_copy(tbl.at[pl.ds(0, BIG)], buf, gsem).wait()
            pltpu.async_copy(buf, out.at[pl.ds(base + g * BIG, BIG)], ssem)
            pltpu.make_async_copy(buf, out.at[pl.ds(0, BIG)], ssem).wait()
            if nxt is not None:
                fire(tbl, idxv, buf, gsem, nxt)

        fire(tsrc_h, si_v, rs0, gsem0, 0)
        fire(tdst_h, di_v, rd0, dsem0, 0)
        fire(tsrc_h, si_v, rs1, gsem1, 1)
        fire(tdst_h, di_v, rd1, dsem1, 1)

        def body(j, carry):
            g = 2 * j
            stage(tsrc_h, si_v, rs0, gsem0, gs_h, ss0, g, g + 2)
            stage(tdst_h, di_v, rd0, dsem0, gd_h, sd0, g, g + 2)
            stage(tsrc_h, si_v, rs1, gsem1, gs_h, ss1, g + 1, g + 3)
            stage(tdst_h, di_v, rd1, dsem1, gd_h, sd1, g + 1, g + 3)
            return carry

        lax.fori_loop(0, (NMAIN - 3) // 2, body, 0)
        stage(tsrc_h, si_v, rs0, gsem0, gs_h, ss0, NMAIN - 3, NMAIN - 1)
        stage(tdst_h, di_v, rd0, dsem0, gd_h, sd0, NMAIN - 3, NMAIN - 1)
        stage(tsrc_h, si_v, rs1, gsem1, gs_h, ss1, NMAIN - 2, None)
        stage(tdst_h, di_v, rd1, dsem1, gd_h, sd1, NMAIN - 2, None)
        pltpu.async_copy(tsrc_h.at[st_v.at[0]], rs1.at[pl.ds(0, tail)], gsem1)
        pltpu.async_copy(tdst_h.at[dt_v.at[0]], rd1.at[pl.ds(0, tail)], dsem1)
        stage(tsrc_h, si_v, rs0, gsem0, gs_h, ss0, NMAIN - 1, None)
        stage(tdst_h, di_v, rd0, dsem0, gd_h, sd0, NMAIN - 1, None)
        tbase = base + NMAIN * BIG
        pltpu.make_async_copy(tsrc_h.at[pl.ds(0, tail)],
                              rs1.at[pl.ds(0, tail)], gsem1).wait()
        pltpu.sync_copy(rs1.at[pl.ds(0, tail)], gs_h.at[pl.ds(tbase, tail)])
        pltpu.make_async_copy(tdst_h.at[pl.ds(0, tail)],
                              rd1.at[pl.ds(0, tail)], dsem1).wait()
        pltpu.sync_copy(rd1.at[pl.ds(0, tail)], gd_h.at[pl.ds(tbase, tail)])

    return k(tsrc, tdst, srcm, srct, dstm, dstt)


def _scatter_call(eupd, dstm, dstt, zeros, epw, tail):
    @functools.partial(
        pl.kernel,
        out_type=jax.ShapeDtypeStruct((NC, NPAD, D), f32),
        mesh=_sc_mesh(),
        scratch_types=[
            pltpu.VMEM((NMAIN, BIG), jnp.int32),
            pltpu.VMEM((1, tail), jnp.int32),
            pltpu.VMEM((BIG, D), f32),
            pltpu.VMEM((BIG, D), f32),
            pltpu.VMEM_SHARED((NPAD, D), f32),
            pltpu.SemaphoreType.DMA,
            pltpu.SemaphoreType.DMA,
            pltpu.SemaphoreType.DMA,
            pltpu.SemaphoreType.DMA,
        ],
    )
    def k(e_h, dstm_h, dstt_h, z_h, parts_h, di_v, dt_v, eb0, eb1, acc_s,
          lsem0, lsem1, asem0, asem1):
        cid = lax.axis_index("c")
        sid = lax.axis_index("s")
        wid = sid * NC + cid
        pltpu.sync_copy(z_h.at[pl.ds(sid * RPT, RPT)],
                        acc_s.at[pl.ds(sid * RPT, RPT)])
        pltpu.sync_copy(dstm_h.at[wid], di_v)
        pltpu.sync_copy(dstt_h.at[wid], dt_v)
        plsc.subcore_barrier()
        base = wid * epw

        def load(buf, sem, g):
            pltpu.async_copy(e_h.at[pl.ds(base + g * BIG, BIG)], buf, sem)

        def stage(buf, lsem, asem, g, nxt):
            pltpu.make_async_copy(e_h.at[pl.ds(0, BIG)], buf, lsem).wait()
            pltpu.async_copy(buf, acc_s.at[di_v.at[g]], asem, add=True)
            pltpu.make_async_copy(buf, acc_s.at[pl.ds(0, BIG)], asem).wait()
            if nxt is not None:
                load(buf, lsem, nxt)

        load(eb0, lsem0, 0)
        load(eb1, lsem1, 1)

        def body(j, carry):
            g = 2 * j
            stage(eb0, lsem0, asem0, g, g + 2)
            stage(eb1, lsem1, asem1, g + 1, g + 3)
            return carry

        lax.fori_loop(0, (NMAIN - 3) // 2, body, 0)
        stage(eb0, lsem0, asem0, NMAIN - 3, NMAIN - 1)
        stage(eb1, lsem1, asem1, NMAIN - 2, None)
        pltpu.async_copy(e_h.at[pl.ds(base + NMAIN * BIG, tail)],
                         eb1.at[pl.ds(0, tail)], lsem1)
        stage(eb0, lsem0, asem0, NMAIN - 1, None)
        pltpu.make_async_copy(e_h.at[pl.ds(0, tail)],
                              eb1.at[pl.ds(0, tail)], lsem1).wait()
        pltpu.async_copy(eb1.at[pl.ds(0, tail)], acc_s.at[dt_v.at[0]],
                         asem1, add=True)
        pltpu.make_async_copy(eb1.at[pl.ds(0, tail)],
                              acc_s.at[pl.ds(0, tail)], asem1).wait()
        plsc.subcore_barrier()
        pltpu.sync_copy(acc_s.at[pl.ds(sid * RPT, RPT)],
                        parts_h.at[cid, pl.ds(sid * RPT, RPT)])

    return k(eupd, dstm, dstt, zeros)


def _unpack(layers, ln):
    lin = layers[:-1] if ln else layers
    ws = []
    for W, b in lin:
        ws += [W, b.reshape(1, -1)]
    if ln:
        g, be = layers[-1]
        ws += [g.reshape(1, -1), be.reshape(1, -1)]
    return ws


def kernel(x, edge_attr, params, edge_index):
    src = edge_index[0]
    dst = edge_index[1]

    def split_idx(v, epw, tail):
        w = v.reshape(NW, epw)
        main = w[:, :NMAIN * BIG].reshape(NW, NMAIN, BIG)
        t = w[:, NMAIN * BIG:].reshape(NW, 1, tail)
        return main, t

    srcAm, srcAt = split_idx(src[:EA], EPWA, TAILA)
    dstAm, dstAt = split_idx(dst[:EA], EPWA, TAILA)
    srcBm, srcBt = split_idx(src[EA:], EPWB, TAILB)
    dstBm, dstBt = split_idx(dst[EA:], EPWB, TAILB)
    zeros = jnp.zeros((NPAD, D), f32)

    h = _row_call(_mlp_ln_body, N_NODES, NODE_BLK,
                  [x] + _unpack(params['enc_node'], True), 1)
    enc_e = _unpack(params['enc_edge'], True)
    eA = _row_call(_mlp_ln_body, EA, BLK_A, [edge_attr[:EA]] + enc_e, 1)
    eB = _row_call(_mlp_ln_body, EB, BLK_B, [edge_attr[EA:]] + enc_e, 1)

    n_steps = len(params['proc'])
    for t, step in enumerate(params['proc']):
        last = t == n_steps - 1
        (W1, b1), (W2, b2), (W3, b3) = step['edge'][:3]
        g, be = step['edge'][3]
        W1e, W1s, W1d = W1[:D], W1[D:2 * D], W1[2 * D:]
        ew = [W1e, b1.reshape(1, -1), W2, b2.reshape(1, -1),
              W3, b3.reshape(1, -1), g.reshape(1, -1), be.reshape(1, -1)]
        ts, td = _row_call(_proj_body, N_NODES, NODE_BLK, [h, W1s, W1d], 2)
        gsA, gdA = _gather_call(ts, td, srcAm, srcAt, dstAm, dstAt, EPWA, TAILA)
        gsB, gdB = _gather_call(ts, td, srcBm, srcBt, dstBm, dstBt, EPWB, TAILB)
        if last:
            uA = _row_call(_edge_body_last, EA, BLK_A, [eA, gsA, gdA] + ew, 1)
            uB = _row_call(_edge_body_last, EB, BLK_B, [eB, gsB, gdB] + ew, 1)
        else:
            uA, eA = _row_call(_edge_body, EA, BLK_A, [eA, gsA, gdA] + ew, 2)
            uB, eB = _row_call(_edge_body, EB, BLK_B, [eB, gsB, gdB] + ew, 2)
        partsA = _scatter_call(uA, dstAm, dstAt, zeros, EPWA, TAILA)
        partsB = _scatter_call(uB, dstBm, dstBt, zeros, EPWB, TAILB)
        (Wn1, nb1), (Wn2, nb2), (Wn3, nb3) = step['node'][:3]
        ng, nbe = step['node'][3]
        h = _row_call(
            _node_body, N_NODES, NODE_BLK,
            [h, partsA[0], partsA[1], partsB[0], partsB[1],
             Wn1[:D], Wn1[D:], nb1.reshape(1, -1),
             Wn2, nb2.reshape(1, -1), Wn3, nb3.reshape(1, -1),
             ng.reshape(1, -1), nbe.reshape(1, -1)], 1)

    (dW1, db1), (dW2, db2), (dW3, db3) = params['dec']
    dW3p = jnp.zeros((D, D), f32).at[:, :3].set(dW3)
    db3p = jnp.zeros((1, D), f32).at[0, :3].set(db3)
    out = _row_call(_dec_body, N_NODES, NODE_BLK,
                    [h, dW1, db1.reshape(1, -1), dW2, db2.reshape(1, -1),
                     dW3p, db3p], 1)
    return out[:, :3]


# speedup vs baseline: 4.4105x; 1.0241x over previous
import functools

import jax
import jax.numpy as jnp
from jax import lax
from jax.experimental import pallas as pl
from jax.experimental.pallas import tpu as pltpu
from jax.experimental.pallas import tpu_sc as plsc

N_NODES = 10000
N_EDGES = 160000
D = 128

NC = 2
NS = 16
NW = NC * NS
BIG = 128
NMAIN = 19
NPAD = 10240
RPT = NPAD // NS

EPWA = 2480
EPWB = 2520
TAILA = EPWA - NMAIN * BIG
TAILB = EPWB - NMAIN * BIG
EA = NW * EPWA
EB = NW * EPWB

NODE_BLK = 1000
BLK_A = EA // 32
BLK_B = EB // 32

f32 = jnp.float32


def _ln(h, g, be):
    mu = jnp.mean(h, axis=-1, keepdims=True)
    d = h - mu
    var = jnp.mean(d * d, axis=-1, keepdims=True)
    return d * lax.rsqrt(var + 1e-5) * g + be


def _mlp_ln_body(x_ref, w1, b1, w2, b2, w3, b3, g, be, out_ref):
    h = jnp.maximum(jnp.dot(x_ref[...], w1[...], preferred_element_type=f32) + b1[...], 0.0)
    h = jnp.maximum(jnp.dot(h, w2[...], preferred_element_type=f32) + b2[...], 0.0)
    h = jnp.dot(h, w3[...], preferred_element_type=f32) + b3[...]
    out_ref[...] = _ln(h, g[...], be[...])


def _mlp_ln_proj_body(x_ref, w1, b1, w2, b2, w3, b3, g, be, ws, wd,
                      out_ref, ts_ref, td_ref):
    h = jnp.maximum(jnp.dot(x_ref[...], w1[...], preferred_element_type=f32) + b1[...], 0.0)
    h = jnp.maximum(jnp.dot(h, w2[...], preferred_element_type=f32) + b2[...], 0.0)
    h = jnp.dot(h, w3[...], preferred_element_type=f32) + b3[...]
    h = _ln(h, g[...], be[...])
    out_ref[...] = h
    ts_ref[...] = jnp.dot(h, ws[...], preferred_element_type=f32)
    td_ref[...] = jnp.dot(h, wd[...], preferred_element_type=f32)


def _edge_mlp(e, gs, gd, w1e, b1, w2, b2, w3, b3, g, be):
    h = jnp.dot(e, w1e[...], preferred_element_type=f32)
    h = jnp.maximum(h + gs + gd + b1[...], 0.0)
    h = jnp.maximum(jnp.dot(h, w2[...], preferred_element_type=f32) + b2[...], 0.0)
    h = jnp.dot(h, w3[...], preferred_element_type=f32) + b3[...]
    return _ln(h, g[...], be[...])


def _edge_body(e_ref, gs_ref, gd_ref, w1e, b1, w2, b2, w3, b3, g, be,
               eupd_ref, enew_ref):
    e = e_ref[...]
    u = _edge_mlp(e, gs_ref[...], gd_ref[...], w1e, b1, w2, b2, w3, b3, g, be)
    eupd_ref[...] = u
    enew_ref[...] = e + u


def _edge_body_last(e_ref, gs_ref, gd_ref, w1e, b1, w2, b2, w3, b3, g, be,
                    eupd_ref):
    u = _edge_mlp(e_ref[...], gs_ref[...], gd_ref[...],
                  w1e, b1, w2, b2, w3, b3, g, be)
    eupd_ref[...] = u


def _node_core(h_ref, pa0_ref, pa1_ref, pb0_ref, pb1_ref,
               wh, wa, b1, w2, b2, w3, b3, g, be):
    h = h_ref[...]
    agg = (pa0_ref[...] + pa1_ref[...]) + (pb0_ref[...] + pb1_ref[...])
    z = jnp.dot(h, wh[...], preferred_element_type=f32)
    z = z + jnp.dot(agg, wa[...], preferred_element_type=f32)
    z = jnp.maximum(z + b1[...], 0.0)
    z = jnp.maximum(jnp.dot(z, w2[...], preferred_element_type=f32) + b2[...], 0.0)
    z = jnp.dot(z, w3[...], preferred_element_type=f32) + b3[...]
    return h + _ln(z, g[...], be[...])


def _node_body(h_ref, pa0_ref, pa1_ref, pb0_ref, pb1_ref,
               wh, wa, b1, w2, b2, w3, b3, g, be, out_ref):
    out_ref[...] = _node_core(h_ref, pa0_ref, pa1_ref, pb0_ref, pb1_ref,
                              wh, wa, b1, w2, b2, w3, b3, g, be)


def _node_proj_body(h_ref, pa0_ref, pa1_ref, pb0_ref, pb1_ref,
                    wh, wa, b1, w2, b2, w3, b3, g, be, ws, wd,
                    out_ref, ts_ref, td_ref):
    h = _node_core(h_ref, pa0_ref, pa1_ref, pb0_ref, pb1_ref,
                   wh, wa, b1, w2, b2, w3, b3, g, be)
    out_ref[...] = h
    ts_ref[...] = jnp.dot(h, ws[...], preferred_element_type=f32)
    td_ref[...] = jnp.dot(h, wd[...], preferred_element_type=f32)


def _dec_body(h_ref, w1, b1, w2, b2, w3, b3, out_ref):
    z = jnp.maximum(jnp.dot(h_ref[...], w1[...], preferred_element_type=f32) + b1[...], 0.0)
    z = jnp.maximum(jnp.dot(z, w2[...], preferred_element_type=f32) + b2[...], 0.0)
    out_ref[...] = jnp.dot(z, w3[...], preferred_element_type=f32) + b3[...]


def _row_call(body, nrows, blk, ins, out_count, out_dim=D, out_dtype=f32):
    def spec(a):
        nd = a.ndim
        if a.shape[0] >= nrows:
            return pl.BlockSpec((blk,) + a.shape[1:],
                                lambda i, nd=nd: (i,) + (0,) * (nd - 1))
        return pl.BlockSpec(a.shape, lambda i, nd=nd: (0,) * nd)

    out_sh = jax.ShapeDtypeStruct((nrows, out_dim), out_dtype)
    out_spec = pl.BlockSpec((blk, out_dim), lambda i: (i, 0))
    return pl.pallas_call(
        body,
        grid=(nrows // blk,),
        in_specs=[spec(a) for a in ins],
        out_specs=[out_spec] * out_count if out_count > 1 else out_spec,
        out_shape=[out_sh] * out_count if out_count > 1 else out_sh,
    )(*ins)


def _sc_mesh():
    return plsc.VectorSubcoreMesh(core_axis_name="c", subcore_axis_name="s")


def _gather_call(tsrc, tdst, srcm, srct, dstm, dstt, epw, tail):
    n_edges = NW * epw

    @functools.partial(
        pl.kernel,
        out_type=(jax.ShapeDtypeStruct((n_edges, D), f32),
                  jax.ShapeDtypeStruct((n_edges, D), f32)),
        mesh=_sc_mesh(),
        scratch_types=[
            pltpu.VMEM((NMAIN, BIG), jnp.int32),
            pltpu.VMEM((NMAIN, BIG), jnp.int32),
            pltpu.VMEM((1, tail), jnp.int32),
            pltpu.VMEM((1, tail), jnp.int32),
            pltpu.VMEM((BIG, D), f32),
            pltpu.VMEM((BIG, D), f32),
            pltpu.VMEM((BIG, D), f32),
            pltpu.VMEM((BIG, D), f32),
            pltpu.SemaphoreType.DMA,
            pltpu.SemaphoreType.DMA,
            pltpu.SemaphoreType.DMA,
            pltpu.SemaphoreType.DMA,
            pltpu.SemaphoreType.DMA,
            pltpu.SemaphoreType.DMA,
            pltpu.SemaphoreType.DMA,
            pltpu.SemaphoreType.DMA,
        ],
    )
    def k(tsrc_h, tdst_h, srcm_h, srct_h, dstm_h, dstt_h, gs_h, gd_h,
          si_v, di_v, st_v, dt_v, rs0, rs1, rd0, rd1,
          gsem0, gsem1, dsem0, dsem1, ss0, ss1, sd0, sd1):
        cid = lax.axis_index("c")
        sid = lax.axis_index("s")
        wid = sid * NC + cid
        pltpu.sync_copy(srcm_h.at[wid], si_v)
        pltpu.sync_copy(dstm_h.at[wid], di_v)
        pltpu.sync_copy(srct_h.at[wid], st_v)
        pltpu.sync_copy(dstt_h.at[wid], dt_v)
        base = wid * epw

        def fire(tbl, idxv, buf, sem, g):
            pltpu.async_copy(tbl.at[idxv.at[g]], buf, sem)

        def stage(tbl, idxv, buf, gsem, out, ssem, g, nxt):
            pltpu.make_async_copy(tbl.at[pl.ds(0, BIG)], buf, gsem).wait()
            pltpu.async_copy(buf, out.at[pl.ds(base + g * BIG, BIG)], ssem)
            pltpu.make_async_copy(buf, out.at[pl.ds(0, BIG)], ssem).wait()
            if nxt is not None:
                fire(tbl, idxv, buf, gsem, nxt)

        fire(tsrc_h, si_v, rs0, gsem0, 0)
        fire(tdst_h, di_v, rd0, dsem0, 0)
        fire(tsrc_h, si_v, rs1, gsem1, 1)
        fire(tdst_h, di_v, rd1, dsem1, 1)

        def body(j, carry):
            g = 2 * j
            stage(tsrc_h, si_v, rs0, gsem0, gs_h, ss0, g, g + 2)
            stage(tdst_h, di_v, rd0, dsem0, gd_h, sd0, g, g + 2)
            stage(tsrc_h, si_v, rs1, gsem1, gs_h, ss1, g + 1, g + 3)
            stage(tdst_h, di_v, rd1, dsem1, gd_h, sd1, g + 1, g + 3)
            return carry

        lax.fori_loop(0, (NMAIN - 3) // 2, body, 0)
        stage(tsrc_h, si_v, rs0, gsem0, gs_h, ss0, NMAIN - 3, NMAIN - 1)
        stage(tdst_h, di_v, rd0, dsem0, gd_h, sd0, NMAIN - 3, NMAIN - 1)
        stage(tsrc_h, si_v, rs1, gsem1, gs_h, ss1, NMAIN - 2, None)
        stage(tdst_h, di_v, rd1, dsem1, gd_h, sd1, NMAIN - 2, None)
        pltpu.async_copy(tsrc_h.at[st_v.at[0]], rs1.at[pl.ds(0, tail)], gsem1)
        pltpu.async_copy(tdst_h.at[dt_v.at[0]], rd1.at[pl.ds(0, tail)], dsem1)
        stage(tsrc_h, si_v, rs0, gsem0, gs_h, ss0, NMAIN - 1, None)
        stage(tdst_h, di_v, rd0, dsem0, gd_h, sd0, NMAIN - 1, None)
        tbase = base + NMAIN * BIG
        pltpu.make_async_copy(tsrc_h.at[pl.ds(0, tail)],
                              rs1.at[pl.ds(0, tail)], gsem1).wait()
        pltpu.sync_copy(rs1.at[pl.ds(0, tail)], gs_h.at[pl.ds(tbase, tail)])
        pltpu.make_async_copy(tdst_h.at[pl.ds(0, tail)],
                              rd1.at[pl.ds(0, tail)], dsem1).wait()
        pltpu.sync_copy(rd1.at[pl.ds(0, tail)], gd_h.at[pl.ds(tbase, tail)])

    return k(tsrc, tdst, srcm, srct, dstm, dstt)


def _scatter_call(eupd, dstm, dstt, zeros, epw, tail):
    @functools.partial(
        pl.kernel,
        out_type=jax.ShapeDtypeStruct((NC, NPAD, D), f32),
        mesh=_sc_mesh(),
        scratch_types=[
            pltpu.VMEM((NMAIN, BIG), jnp.int32),
            pltpu.VMEM((1, tail), jnp.int32),
            pltpu.VMEM((BIG, D), f32),
            pltpu.VMEM((BIG, D), f32),
            pltpu.VMEM_SHARED((NPAD, D), f32),
            pltpu.SemaphoreType.DMA,
            pltpu.SemaphoreType.DMA,
            pltpu.SemaphoreType.DMA,
            pltpu.SemaphoreType.DMA,
        ],
    )
    def k(e_h, dstm_h, dstt_h, z_h, parts_h, di_v, dt_v, eb0, eb1, acc_s,
          lsem0, lsem1, asem0, asem1):
        cid = lax.axis_index("c")
        sid = lax.axis_index("s")
        wid = sid * NC + cid
        pltpu.sync_copy(z_h.at[pl.ds(sid * RPT, RPT)],
                        acc_s.at[pl.ds(sid * RPT, RPT)])
        pltpu.sync_copy(dstm_h.at[wid], di_v)
        pltpu.sync_copy(dstt_h.at[wid], dt_v)
        plsc.subcore_barrier()
        base = wid * epw

        def load(buf, sem, g):
            pltpu.async_copy(e_h.at[pl.ds(base + g * BIG, BIG)], buf, sem)

        def stage(buf, lsem, asem, g, nxt):
            pltpu.make_async_copy(e_h.at[pl.ds(0, BIG)], buf, lsem).wait()
            pltpu.async_copy(buf, acc_s.at[di_v.at[g]], asem, add=True)
            pltpu.make_async_copy(buf, acc_s.at[pl.ds(0, BIG)], asem).wait()
            if nxt is not None:
                load(buf, lsem, nxt)

        load(eb0, lsem0, 0)
        load(eb1, lsem1, 1)

        def body(j, carry):
            g = 2 * j
            stage(eb0, lsem0, asem0, g, g + 2)
            stage(eb1, lsem1, asem1, g + 1, g + 3)
            return carry

        lax.fori_loop(0, (NMAIN - 3) // 2, body, 0)
        stage(eb0, lsem0, asem0, NMAIN - 3, NMAIN - 1)
        stage(eb1, lsem1, asem1, NMAIN - 2, None)
        pltpu.async_copy(e_h.at[pl.ds(base + NMAIN * BIG, tail)],
                         eb1.at[pl.ds(0, tail)], lsem1)
        stage(eb0, lsem0, asem0, NMAIN - 1, None)
        pltpu.make_async_copy(e_h.at[pl.ds(0, tail)],
                              eb1.at[pl.ds(0, tail)], lsem1).wait()
        pltpu.async_copy(eb1.at[pl.ds(0, tail)], acc_s.at[dt_v.at[0]],
                         asem1, add=True)
        pltpu.make_async_copy(eb1.at[pl.ds(0, tail)],
                              acc_s.at[pl.ds(0, tail)], asem1).wait()
        plsc.subcore_barrier()
        pltpu.sync_copy(acc_s.at[pl.ds(sid * RPT, RPT)],
                        parts_h.at[cid, pl.ds(sid * RPT, RPT)])

    return k(eupd, dstm, dstt, zeros)


def _unpack(layers, ln):
    lin = layers[:-1] if ln else layers
    ws = []
    for W, b in lin:
        ws += [W, b.reshape(1, -1)]
    if ln:
        g, be = layers[-1]
        ws += [g.reshape(1, -1), be.reshape(1, -1)]
    return ws


def kernel(x, edge_attr, params, edge_index):
    src = edge_index[0]
    dst = edge_index[1]

    def split_idx(v, epw, tail):
        w = v.reshape(NW, epw)
        main = w[:, :NMAIN * BIG].reshape(NW, NMAIN, BIG)
        t = w[:, NMAIN * BIG:].reshape(NW, 1, tail)
        return main, t

    srcAm, srcAt = split_idx(src[:EA], EPWA, TAILA)
    dstAm, dstAt = split_idx(dst[:EA], EPWA, TAILA)
    srcBm, srcBt = split_idx(src[EA:], EPWB, TAILB)
    dstBm, dstBt = split_idx(dst[EA:], EPWB, TAILB)
    zeros = jnp.zeros((NPAD, D), f32)

    steps = []
    for step in params['proc']:
        (W1, b1), (W2, b2), (W3, b3) = step['edge'][:3]
        g, be = step['edge'][3]
        (Wn1, nb1), (Wn2, nb2), (Wn3, nb3) = step['node'][:3]
        ng, nbe = step['node'][3]
        steps.append(dict(
            W1s=W1[D:2 * D], W1d=W1[2 * D:],
            ew=[W1[:D], b1.reshape(1, -1), W2, b2.reshape(1, -1),
                W3, b3.reshape(1, -1), g.reshape(1, -1), be.reshape(1, -1)],
            nw=[Wn1[:D], Wn1[D:], nb1.reshape(1, -1),
                Wn2, nb2.reshape(1, -1), Wn3, nb3.reshape(1, -1),
                ng.reshape(1, -1), nbe.reshape(1, -1)]))

    h, ts, td = _row_call(
        _mlp_ln_proj_body, N_NODES, NODE_BLK,
        [x] + _unpack(params['enc_node'], True)
        + [steps[0]['W1s'], steps[0]['W1d']], 3)
    enc_e = _unpack(params['enc_edge'], True)
    eA = _row_call(_mlp_ln_body, EA, BLK_A, [edge_attr[:EA]] + enc_e, 1)
    eB = _row_call(_mlp_ln_body, EB, BLK_B, [edge_attr[EA:]] + enc_e, 1)

    n_steps = len(steps)
    for t, st in enumerate(steps):
        last = t == n_steps - 1
        gsA, gdA = _gather_call(ts, td, srcAm, srcAt, dstAm, dstAt, EPWA, TAILA)
        gsB, gdB = _gather_call(ts, td, srcBm, srcBt, dstBm, dstBt, EPWB, TAILB)
        ew = st['ew']
        if last:
            uA = _row_call(_edge_body_last, EA, BLK_A, [eA, gsA, gdA] + ew, 1)
            uB = _row_call(_edge_body_last, EB, BLK_B, [eB, gsB, gdB] + ew, 1)
        else:
            uA, eA = _row_call(_edge_body, EA, BLK_A, [eA, gsA, gdA] + ew, 2)
            uB, eB = _row_call(_edge_body, EB, BLK_B, [eB, gsB, gdB] + ew, 2)
        partsA = _scatter_call(uA, dstAm, dstAt, zeros, EPWA, TAILA)
        partsB = _scatter_call(uB, dstBm, dstBt, zeros, EPWB, TAILB)
        nin = [h, partsA[0], partsA[1], partsB[0], partsB[1]] + st['nw']
        if last:
            h = _row_call(_node_body, N_NODES, NODE_BLK, nin, 1)
        else:
            nxt = steps[t + 1]
            h, ts, td = _row_call(_node_proj_body, N_NODES, NODE_BLK,
                                  nin + [nxt['W1s'], nxt['W1d']], 3)

    (dW1, db1), (dW2, db2), (dW3, db3) = params['dec']
    dW3p = jnp.zeros((D, D), f32).at[:, :3].set(dW3)
    db3p = jnp.zeros((1, D), f32).at[0, :3].set(db3)
    out = _row_call(_dec_body, N_NODES, NODE_BLK,
                    [h, dW1, db1.reshape(1, -1), dW2, db2.reshape(1, -1),
                     dW3p, db3p], 1)
    return out[:, :3]


# speedup vs baseline: 4.4719x; 1.0139x over previous
import functools

import jax
import jax.numpy as jnp
from jax import lax
from jax.experimental import pallas as pl
from jax.experimental.pallas import tpu as pltpu
from jax.experimental.pallas import tpu_sc as plsc

N_NODES = 10000
N_EDGES = 160000
D = 128

NC = 2
NS = 16
NW = NC * NS
BIG = 128
NMAIN = 19
NPAD = 10240
RPT = NPAD // NS

EPWA = 2480
EPWB = 2520
TAILA = EPWA - NMAIN * BIG
TAILB = EPWB - NMAIN * BIG
EA = NW * EPWA
EB = NW * EPWB

NODE_BLK = 1000
BLK_A = EA // 32
BLK_B = EB // 32

f32 = jnp.float32


def _ln(h, g, be):
    mu = jnp.mean(h, axis=-1, keepdims=True)
    d = h - mu
    var = jnp.mean(d * d, axis=-1, keepdims=True)
    return d * lax.rsqrt(var + 1e-5) * g + be


def _mlp_ln_body(x_ref, w1, b1, w2, b2, w3, b3, g, be, out_ref):
    h = jnp.maximum(jnp.dot(x_ref[...], w1[...], preferred_element_type=f32) + b1[...], 0.0)
    h = jnp.maximum(jnp.dot(h, w2[...], preferred_element_type=f32) + b2[...], 0.0)
    h = jnp.dot(h, w3[...], preferred_element_type=f32) + b3[...]
    out_ref[...] = _ln(h, g[...], be[...])


def _mlp_ln_proj_body(x_ref, w1, b1, w2, b2, w3, b3, g, be, ws, wd,
                      out_ref, ts_ref, td_ref):
    h = jnp.maximum(jnp.dot(x_ref[...], w1[...], preferred_element_type=f32) + b1[...], 0.0)
    h = jnp.maximum(jnp.dot(h, w2[...], preferred_element_type=f32) + b2[...], 0.0)
    h = jnp.dot(h, w3[...], preferred_element_type=f32) + b3[...]
    h = _ln(h, g[...], be[...])
    out_ref[...] = h
    ts_ref[...] = jnp.dot(h, ws[...], preferred_element_type=f32)
    td_ref[...] = jnp.dot(h, wd[...], preferred_element_type=f32)


def _edge_mlp(e, gs, gd, w1e, b1, w2, b2, w3, b3, g, be):
    h = jnp.dot(e, w1e[...], preferred_element_type=f32)
    h = jnp.maximum(h + gs + gd + b1[...], 0.0)
    h = jnp.maximum(jnp.dot(h, w2[...], preferred_element_type=f32) + b2[...], 0.0)
    h = jnp.dot(h, w3[...], preferred_element_type=f32) + b3[...]
    return _ln(h, g[...], be[...])


def _edge_body(e_ref, gs_ref, gd_ref, w1e, b1, w2, b2, w3, b3, g, be,
               eupd_ref, enew_ref):
    e = e_ref[...]
    u = _edge_mlp(e, gs_ref[...], gd_ref[...], w1e, b1, w2, b2, w3, b3, g, be)
    eupd_ref[...] = u
    enew_ref[...] = e + u


def _edge_body_last(e_ref, gs_ref, gd_ref, w1e, b1, w2, b2, w3, b3, g, be,
                    eupd_ref):
    u = _edge_mlp(e_ref[...], gs_ref[...], gd_ref[...],
                  w1e, b1, w2, b2, w3, b3, g, be)
    eupd_ref[...] = u


def _node_core(h_ref, pa0_ref, pa1_ref, pb0_ref, pb1_ref,
               wh, wa, b1, w2, b2, w3, b3, g, be):
    h = h_ref[...]
    agg = (pa0_ref[...] + pa1_ref[...]) + (pb0_ref[...] + pb1_ref[...])
    z = jnp.dot(h, wh[...], preferred_element_type=f32)
    z = z + jnp.dot(agg, wa[...], preferred_element_type=f32)
    z = jnp.maximum(z + b1[...], 0.0)
    z = jnp.maximum(jnp.dot(z, w2[...], preferred_element_type=f32) + b2[...], 0.0)
    z = jnp.dot(z, w3[...], preferred_element_type=f32) + b3[...]
    return h + _ln(z, g[...], be[...])


def _node_body(h_ref, pa0_ref, pa1_ref, pb0_ref, pb1_ref,
               wh, wa, b1, w2, b2, w3, b3, g, be, out_ref):
    out_ref[...] = _node_core(h_ref, pa0_ref, pa1_ref, pb0_ref, pb1_ref,
                              wh, wa, b1, w2, b2, w3, b3, g, be)


def _node_proj_body(h_ref, pa0_ref, pa1_ref, pb0_ref, pb1_ref,
                    wh, wa, b1, w2, b2, w3, b3, g, be, ws, wd,
                    out_ref, ts_ref, td_ref):
    h = _node_core(h_ref, pa0_ref, pa1_ref, pb0_ref, pb1_ref,
                   wh, wa, b1, w2, b2, w3, b3, g, be)
    out_ref[...] = h
    ts_ref[...] = jnp.dot(h, ws[...], preferred_element_type=f32)
    td_ref[...] = jnp.dot(h, wd[...], preferred_element_type=f32)


def _dec_body(h_ref, w1, b1, w2, b2, w3, b3, out_ref):
    z = jnp.maximum(jnp.dot(h_ref[...], w1[...], preferred_element_type=f32) + b1[...], 0.0)
    z = jnp.maximum(jnp.dot(z, w2[...], preferred_element_type=f32) + b2[...], 0.0)
    out_ref[...] = jnp.dot(z, w3[...], preferred_element_type=f32) + b3[...]


def _row_call(body, nrows, blk, ins, out_count, out_dim=D, out_dtype=f32):
    def spec(a):
        nd = a.ndim
        if a.shape[0] >= nrows:
            return pl.BlockSpec((blk,) + a.shape[1:],
                                lambda i, nd=nd: (i,) + (0,) * (nd - 1))
        return pl.BlockSpec(a.shape, lambda i, nd=nd: (0,) * nd)

    out_sh = jax.ShapeDtypeStruct((nrows, out_dim), out_dtype)
    out_spec = pl.BlockSpec((blk, out_dim), lambda i: (i, 0))
    return pl.pallas_call(
        body,
        grid=(nrows // blk,),
        in_specs=[spec(a) for a in ins],
        out_specs=[out_spec] * out_count if out_count > 1 else out_spec,
        out_shape=[out_sh] * out_count if out_count > 1 else out_sh,
    )(*ins)


def _sc_mesh():
    return plsc.VectorSubcoreMesh(core_axis_name="c", subcore_axis_name="s")


def _gather_call(tsrc, tdst, srcm, srct, dstm, dstt, epw, tail):
    n_edges = NW * epw

    @functools.partial(
        pl.kernel,
        out_type=(jax.ShapeDtypeStruct((n_edges, D), f32),
                  jax.ShapeDtypeStruct((n_edges, D), f32)),
        mesh=_sc_mesh(),
        scratch_types=[
            pltpu.VMEM((NMAIN, BIG), jnp.int32),
            pltpu.VMEM((NMAIN, BIG), jnp.int32),
            pltpu.VMEM((1, tail), jnp.int32),
            pltpu.VMEM((1, tail), jnp.int32),
            pltpu.VMEM((BIG, D), f32),
            pltpu.VMEM((BIG, D), f32),
            pltpu.VMEM((BIG, D), f32),
            pltpu.VMEM((BIG, D), f32),
            pltpu.SemaphoreType.DMA,
            pltpu.SemaphoreType.DMA,
            pltpu.SemaphoreType.DMA,
            pltpu.SemaphoreType.DMA,
            pltpu.SemaphoreType.DMA,
            pltpu.SemaphoreType.DMA,
            pltpu.SemaphoreType.DMA,
            pltpu.SemaphoreType.DMA,
        ],
    )
    def k(tsrc_h, tdst_h, srcm_h, srct_h, dstm_h, dstt_h, gs_h, gd_h,
          si_v, di_v, st_v, dt_v, rs0, rs1, rd0, rd1,
          gsem0, gsem1, dsem0, dsem1, ss0, ss1, sd0, sd1):
        cid = lax.axis_index("c")
        sid = lax.axis_index("s")
        wid = sid * NC + cid
        pltpu.sync_copy(srcm_h.at[wid], si_v)
        pltpu.sync_copy(dstm_h.at[wid], di_v)
        pltpu.sync_copy(srct_h.at[wid], st_v)
        pltpu.sync_copy(dstt_h.at[wid], dt_v)
        base = wid * epw

        def fire(tbl, idxv, buf, sem, g):
            pltpu.async_copy(tbl.at[idxv.at[g]], buf, sem)

        def stage(tbl, idxv, buf, gsem, out, ssem, g, nxt):
            pltpu.make_async_copy(tbl.at[pl.ds(0, BIG)], buf, gsem).wait()
            pltpu.async_copy(buf, out.at[pl.ds(base + g * BIG, BIG)], ssem)
            pltpu.make_async_copy(buf, out.at[pl.ds(0, BIG)], ssem).wait()
            if nxt is not None:
                fire(tbl, idxv, buf, gsem, nxt)

        fire(tsrc_h, si_v, rs0, gsem0, 0)
        fire(tdst_h, di_v, rd0, dsem0, 0)
        fire(tsrc_h, si_v, rs1, gsem1, 1)
        fire(tdst_h, di_v, rd1, dsem1, 1)

        def body(j, carry):
            g = 2 * j
            stage(tsrc_h, si_v, rs0, gsem0, gs_h, ss0, g, g + 2)
            stage(tdst_h, di_v, rd0, dsem0, gd_h, sd0, g, g + 2)
            stage(tsrc_h, si_v, rs1, gsem1, gs_h, ss1, g + 1, g + 3)
            stage(tdst_h, di_v, rd1, dsem1, gd_h, sd1, g + 1, g + 3)
            return carry

        lax.fori_loop(0, (NMAIN - 3) // 2, body, 0)
        stage(tsrc_h, si_v, rs0, gsem0, gs_h, ss0, NMAIN - 3, NMAIN - 1)
        stage(tdst_h, di_v, rd0, dsem0, gd_h, sd0, NMAIN - 3, NMAIN - 1)
        stage(tsrc_h, si_v, rs1, gsem1, gs_h, ss1, NMAIN - 2, None)
        stage(tdst_h, di_v, rd1, dsem1, gd_h, sd1, NMAIN - 2, None)
        pltpu.async_copy(tsrc_h.at[st_v.at[0]], rs1.at[pl.ds(0, tail)], gsem1)
        pltpu.async_copy(tdst_h.at[dt_v.at[0]], rd1.at[pl.ds(0, tail)], dsem1)
        stage(tsrc_h, si_v, rs0, gsem0, gs_h, ss0, NMAIN - 1, None)
        stage(tdst_h, di_v, rd0, dsem0, gd_h, sd0, NMAIN - 1, None)
        tbase = base + NMAIN * BIG
        pltpu.make_async_copy(tsrc_h.at[pl.ds(0, tail)],
                              rs1.at[pl.ds(0, tail)], gsem1).wait()
        pltpu.sync_copy(rs1.at[pl.ds(0, tail)], gs_h.at[pl.ds(tbase, tail)])
        pltpu.make_async_copy(tdst_h.at[pl.ds(0, tail)],
                              rd1.at[pl.ds(0, tail)], dsem1).wait()
        pltpu.sync_copy(rd1.at[pl.ds(0, tail)], gd_h.at[pl.ds(tbase, tail)])

    return k(tsrc, tdst, srcm, srct, dstm, dstt)


ZR = 64


def _scatter_call(eupd, dstm, dstt, epw, tail):
    @functools.partial(
        pl.kernel,
        out_type=jax.ShapeDtypeStruct((NC, NPAD, D), f32),
        mesh=_sc_mesh(),
        scratch_types=[
            pltpu.VMEM((NMAIN, BIG), jnp.int32),
            pltpu.VMEM((1, tail), jnp.int32),
            pltpu.VMEM((BIG, D), f32),
            pltpu.VMEM((BIG, D), f32),
            pltpu.VMEM((ZR, D), f32),
            pltpu.VMEM_SHARED((NPAD, D), f32),
            pltpu.SemaphoreType.DMA,
            pltpu.SemaphoreType.DMA,
            pltpu.SemaphoreType.DMA,
            pltpu.SemaphoreType.DMA,
        ],
    )
    def k(e_h, dstm_h, dstt_h, parts_h, di_v, dt_v, eb0, eb1, zb, acc_s,
          lsem0, lsem1, asem0, asem1):
        cid = lax.axis_index("c")
        sid = lax.axis_index("s")
        wid = sid * NC + cid
        pltpu.sync_copy(dstm_h.at[wid], di_v)
        pltpu.sync_copy(dstt_h.at[wid], dt_v)

        def zrow(r, carry):
            for kk in range(D // 16):
                zb[r, pl.ds(kk * 16, 16)] = jnp.zeros((16,), f32)
            return carry

        lax.fori_loop(0, ZR, zrow, 0)
        for i in range(RPT // ZR):
            pltpu.sync_copy(zb, acc_s.at[pl.ds(sid * RPT + i * ZR, ZR)])
        plsc.subcore_barrier()
        base = wid * epw

        def load(buf, sem, g):
            pltpu.async_copy(e_h.at[pl.ds(base + g * BIG, BIG)], buf, sem)

        def stage(buf, lsem, asem, g, nxt):
            pltpu.make_async_copy(e_h.at[pl.ds(0, BIG)], buf, lsem).wait()
            pltpu.async_copy(buf, acc_s.at[di_v.at[g]], asem, add=True)
            pltpu.make_async_copy(buf, acc_s.at[pl.ds(0, BIG)], asem).wait()
            if nxt is not None:
                load(buf, lsem, nxt)

        load(eb0, lsem0, 0)
        load(eb1, lsem1, 1)

        def body(j, carry):
            g = 2 * j
            stage(eb0, lsem0, asem0, g, g + 2)
            stage(eb1, lsem1, asem1, g + 1, g + 3)
            return carry

        lax.fori_loop(0, (NMAIN - 3) // 2, body, 0)
        stage(eb0, lsem0, asem0, NMAIN - 3, NMAIN - 1)
        stage(eb1, lsem1, asem1, NMAIN - 2, None)
        pltpu.async_copy(e_h.at[pl.ds(base + NMAIN * BIG, tail)],
                         eb1.at[pl.ds(0, tail)], lsem1)
        stage(eb0, lsem0, asem0, NMAIN - 1, None)
        pltpu.make_async_copy(e_h.at[pl.ds(0, tail)],
                              eb1.at[pl.ds(0, tail)], lsem1).wait()
        pltpu.async_copy(eb1.at[pl.ds(0, tail)], acc_s.at[dt_v.at[0]],
                         asem1, add=True)
        pltpu.make_async_copy(eb1.at[pl.ds(0, tail)],
                              acc_s.at[pl.ds(0, tail)], asem1).wait()
        plsc.subcore_barrier()
        pltpu.sync_copy(acc_s.at[pl.ds(sid * RPT, RPT)],
                        parts_h.at[cid, pl.ds(sid * RPT, RPT)])

    return k(eupd, dstm, dstt)


def _unpack(layers, ln):
    lin = layers[:-1] if ln else layers
    ws = []
    for W, b in lin:
        ws += [W, b.reshape(1, -1)]
    if ln:
        g, be = layers[-1]
        ws += [g.reshape(1, -1), be.reshape(1, -1)]
    return ws


def kernel(x, edge_attr, params, edge_index):
    src = edge_index[0]
    dst = edge_index[1]

    def split_idx(v, epw, tail):
        w = v.reshape(NW, epw)
        main = w[:, :NMAIN * BIG].reshape(NW, NMAIN, BIG)
        t = w[:, NMAIN * BIG:].reshape(NW, 1, tail)
        return main, t

    srcAm, srcAt = split_idx(src[:EA], EPWA, TAILA)
    dstAm, dstAt = split_idx(dst[:EA], EPWA, TAILA)
    srcBm, srcBt = split_idx(src[EA:], EPWB, TAILB)
    dstBm, dstBt = split_idx(dst[EA:], EPWB, TAILB)

    steps = []
    for step in params['proc']:
        (W1, b1), (W2, b2), (W3, b3) = step['edge'][:3]
        g, be = step['edge'][3]
        (Wn1, nb1), (Wn2, nb2), (Wn3, nb3) = step['node'][:3]
        ng, nbe = step['node'][3]
        steps.append(dict(
            W1s=W1[D:2 * D], W1d=W1[2 * D:],
            ew=[W1[:D], b1.reshape(1, -1), W2, b2.reshape(1, -1),
                W3, b3.reshape(1, -1), g.reshape(1, -1), be.reshape(1, -1)],
            nw=[Wn1[:D], Wn1[D:], nb1.reshape(1, -1),
                Wn2, nb2.reshape(1, -1), Wn3, nb3.reshape(1, -1),
                ng.reshape(1, -1), nbe.reshape(1, -1)]))

    h, ts, td = _row_call(
        _mlp_ln_proj_body, N_NODES, NODE_BLK,
        [x] + _unpack(params['enc_node'], True)
        + [steps[0]['W1s'], steps[0]['W1d']], 3)
    enc_e = _unpack(params['enc_edge'], True)
    eA = _row_call(_mlp_ln_body, EA, BLK_A, [edge_attr[:EA]] + enc_e, 1)
    eB = _row_call(_mlp_ln_body, EB, BLK_B, [edge_attr[EA:]] + enc_e, 1)

    n_steps = len(steps)
    for t, st in enumerate(steps):
        last = t == n_steps - 1
        gsA, gdA = _gather_call(ts, td, srcAm, srcAt, dstAm, dstAt, EPWA, TAILA)
        gsB, gdB = _gather_call(ts, td, srcBm, srcBt, dstBm, dstBt, EPWB, TAILB)
        ew = st['ew']
        if last:
            uA = _row_call(_edge_body_last, EA, BLK_A, [eA, gsA, gdA] + ew, 1)
            uB = _row_call(_edge_body_last, EB, BLK_B, [eB, gsB, gdB] + ew, 1)
        else:
            uA, eA = _row_call(_edge_body, EA, BLK_A, [eA, gsA, gdA] + ew, 2)
            uB, eB = _row_call(_edge_body, EB, BLK_B, [eB, gsB, gdB] + ew, 2)
        partsA = _scatter_call(uA, dstAm, dstAt, EPWA, TAILA)
        partsB = _scatter_call(uB, dstBm, dstBt, EPWB, TAILB)
        nin = [h, partsA[0], partsA[1], partsB[0], partsB[1]] + st['nw']
        if last:
            h = _row_call(_node_body, N_NODES, NODE_BLK, nin, 1)
        else:
            nxt = steps[t + 1]
            h, ts, td = _row_call(_node_proj_body, N_NODES, NODE_BLK,
                                  nin + [nxt['W1s'], nxt['W1d']], 3)

    (dW1, db1), (dW2, db2), (dW3, db3) = params['dec']
    dW3p = jnp.zeros((D, D), f32).at[:, :3].set(dW3)
    db3p = jnp.zeros((1, D), f32).at[0, :3].set(db3)
    out = _row_call(_dec_body, N_NODES, NODE_BLK,
                    [h, dW1, db1.reshape(1, -1), dW2, db2.reshape(1, -1),
                     dW3p, db3p], 1)
    return out[:, :3]
